# baseline (speedup 1.0000x reference)
import functools

import jax
import jax.numpy as jnp
from jax import lax
from jax.experimental import pallas as pl
from jax.experimental.pallas import tpu as pltpu
from jax.experimental.pallas import tpu_sc as plsc

D_MODEL = 64
NUM_CORES = 2
NUM_SUBCORES = 16
NUM_WORKERS = NUM_CORES * NUM_SUBCORES


@functools.lru_cache(maxsize=None)
def _make_kernel(n_rows: int, chunk: int):
    assert n_rows % (NUM_WORKERS * chunk) == 0
    rows_per_w = n_rows // NUM_WORKERS
    n_chunks = rows_per_w // chunk
    mesh = plsc.VectorSubcoreMesh(core_axis_name="c", subcore_axis_name="s")

    @functools.partial(
        pl.kernel,
        mesh=mesh,
        compiler_params=pltpu.CompilerParams(use_tc_tiling_on_sc=False),
        out_type=jax.ShapeDtypeStruct((n_rows, D_MODEL), jnp.float32),
        scratch_types=[
            pltpu.VMEM((chunk,), jnp.int32),
            pltpu.VMEM((chunk, D_MODEL), jnp.float32),
            pltpu.SemaphoreType.DMA,
        ],
    )
    def k(idx_hbm, table_hbm, out_hbm, idx_v, rows_v, sem):
        wid = lax.axis_index("s") * NUM_CORES + lax.axis_index("c")
        base_w = wid * rows_per_w

        def body(g, carry):
            base = pl.multiple_of(base_w + g * chunk, 8)
            pltpu.sync_copy(idx_hbm.at[pl.ds(base, chunk)], idx_v)
            pltpu.async_copy(table_hbm.at[idx_v], rows_v, sem).wait()
            pltpu.sync_copy(rows_v, out_hbm.at[pl.ds(base, chunk)])
            return carry

        lax.fori_loop(0, n_chunks, body, 0)

    return k


def kernel(x, embed_weight):
    b0, b1 = x.shape
    flat_idx = x.reshape(-1).astype(jnp.int32)
    out = _make_kernel(b0 * b1, 800)(flat_idx, embed_weight)
    return out.reshape(b0, b1, D_MODEL)


# speedup vs baseline: 1.0485x; 1.0485x over previous
import functools

import jax
import jax.numpy as jnp
from jax import lax
from jax.experimental import pallas as pl
from jax.experimental.pallas import tpu as pltpu
from jax.experimental.pallas import tpu_sc as plsc

D_MODEL = 64
NUM_CORES = 2
NUM_SUBCORES = 16
NUM_WORKERS = NUM_CORES * NUM_SUBCORES


@functools.lru_cache(maxsize=None)
def _make_kernel(n_rows: int, chunk: int):
    assert n_rows % (NUM_WORKERS * chunk) == 0
    rows_per_w = n_rows // NUM_WORKERS
    n_chunks = rows_per_w // chunk
    assert n_chunks % 2 == 0 and n_chunks >= 4
    half_t = n_chunks // 2
    mesh = plsc.VectorSubcoreMesh(core_axis_name="c", subcore_axis_name="s")

    @functools.partial(
        pl.kernel,
        mesh=mesh,
        compiler_params=pltpu.CompilerParams(use_tc_tiling_on_sc=False),
        out_type=jax.ShapeDtypeStruct((n_rows, D_MODEL), jnp.float32),
        scratch_types=[
            pltpu.VMEM((chunk,), jnp.int32),
            pltpu.VMEM((chunk,), jnp.int32),
            pltpu.VMEM((chunk, D_MODEL), jnp.float32),
            pltpu.VMEM((chunk, D_MODEL), jnp.float32),
            pltpu.SemaphoreType.DMA,
            pltpu.SemaphoreType.DMA,
            pltpu.SemaphoreType.DMA,
            pltpu.SemaphoreType.DMA,
            pltpu.SemaphoreType.DMA,
            pltpu.SemaphoreType.DMA,
        ],
    )
    def k(idx_hbm, table_hbm, out_hbm, idx0, idx1, rows0, rows1,
          si0, si1, sg0, sg1, so0, so1):
        wid = lax.axis_index("s") * NUM_CORES + lax.axis_index("c")
        base_w = wid * rows_per_w
        idx_v = (idx0, idx1)
        rows_v = (rows0, rows1)
        sem_i = (si0, si1)
        sem_g = (sg0, sg1)
        sem_o = (so0, so1)

        def hbm_slice(ref, c):
            return ref.at[pl.ds(pl.multiple_of(base_w + c * chunk, 8), chunk)]

        def start_idx(c, b):
            pltpu.async_copy(hbm_slice(idx_hbm, c), idx_v[b], sem_i[b])

        def gather_desc(b):
            return pltpu.make_async_copy(
                table_hbm.at[idx_v[b]], rows_v[b], sem_g[b])

        def wb_desc(c, b):
            return pltpu.make_async_copy(
                rows_v[b], hbm_slice(out_hbm, c), sem_o[b])

        def idx_wait(b):
            pltpu.make_async_copy(
                idx_hbm.at[pl.ds(0, chunk)], idx_v[b], sem_i[b]).wait()

        def slot(c, b, *, wait_out, start_next_idx):
            ob = 1 - b
            idx_wait(b)
            if wait_out:
                wb_desc(c, b).wait()
            gather_desc(b).start()
            gather_desc(ob).wait()
            wb_desc(c - 1, ob).start()
            if start_next_idx:
                start_idx(c + 1, ob)

        start_idx(0, 0)
        idx_wait(0)
        gather_desc(0).start()
        start_idx(1, 1)
        idx_wait(1)
        gather_desc(1).start()
        gather_desc(0).wait()
        wb_desc(0, 0).start()
        start_idx(2, 0)

        def body(t, carry):
            c = t * 2
            slot(c, 0, wait_out=True, start_next_idx=True)
            slot(c + 1, 1, wait_out=True, start_next_idx=True)
            return carry

        lax.fori_loop(1, half_t - 1, body, 0)

        c = n_chunks - 2
        slot(c, 0, wait_out=True, start_next_idx=True)
        slot(c + 1, 1, wait_out=True, start_next_idx=False)
        gather_desc(1).wait()
        wb_desc(n_chunks - 1, 1).start()
        wb_desc(n_chunks - 2, 0).wait()
        wb_desc(n_chunks - 1, 1).wait()

    return k


def kernel(x, embed_weight):
    b0, b1 = x.shape
    flat_idx = x.reshape(-1).astype(jnp.int32)
    out = _make_kernel(b0 * b1, 800)(flat_idx, embed_weight)
    return out.reshape(b0, b1, D_MODEL)
